# baseline (speedup 1.0000x reference)
import functools

import jax
import jax.numpy as jnp
from jax import lax
from jax.experimental import pallas as pl
from jax.experimental.pallas import tpu as pltpu
from jax.experimental.pallas import tpu_sc as plsc

N = 100000
DA = 128
L = 16
NC = 2
NS = 16
NW = NC * NS
R = 128
NFULL = N // R
TAIL = N - NFULL * R
TAIL_BASE = NFULL * R
BIG_WORKERS = NFULL % NW
K = DA // L


def _body(x_hbm, v_hbm, u_hbm, b_hbm, beff_hbm, eff_hbm, y_hbm,
          xin, yout, effb, vvm, uvm, bvm, beffvm):
    wid = lax.axis_index("s") * NC + lax.axis_index("c")

    pltpu.sync_copy(v_hbm, vvm)
    pltpu.sync_copy(u_hbm, uvm)
    pltpu.sync_copy(b_hbm, bvm)
    pltpu.sync_copy(beff_hbm, beffvm)

    vv = [vvm[pl.ds(k * L, L)] for k in range(K)]
    uu = [uvm[pl.ds(k * L, L)] for k in range(K)]
    bb = [bvm[pl.ds(k * L, L)] for k in range(K)]
    beffv = beffvm[...]
    lane = lax.iota(jnp.int32, L)

    def do_rows(nrows):
        for g in range(nrows // L):
            eff16 = jnp.zeros((L,), jnp.float32)
            for i in range(L):
                r = g * L + i
                xs = [xin[r, pl.ds(k * L, L)] for k in range(K)]
                ps = [xs[k] * vv[k] for k in range(K)]
                while len(ps) > 1:
                    ps = [ps[j] + ps[j + 1] for j in range(0, len(ps), 2)]
                zs = jnp.sum(ps[0])
                zv = zs + beffv
                eff = 1.0 / (1.0 + jnp.exp(-zv))
                eff16 = jnp.where(lane == i, eff, eff16)
                for k in range(K):
                    yout[r, pl.ds(k * L, L)] = eff * (uu[k] * xs[k] + bb[k])
            effb[pl.ds(g * L, L)] = eff16

    nchunks = 25 - (wid >= BIG_WORKERS).astype(jnp.int32)

    def chunk_body(t, carry):
        c = wid + t * NW
        base = c * R
        pltpu.sync_copy(x_hbm.at[pl.ds(base, R)], xin)
        do_rows(R)
        pltpu.sync_copy(yout, y_hbm.at[pl.ds(base, R)])
        pltpu.sync_copy(effb, eff_hbm.at[pl.ds(base, R)])
        return carry

    lax.fori_loop(0, nchunks, chunk_body, 0)

    @pl.when(wid == NW - 1)
    def _tail():
        pltpu.sync_copy(x_hbm.at[pl.ds(TAIL_BASE, TAIL)], xin.at[pl.ds(0, TAIL)])
        do_rows(TAIL)
        pltpu.sync_copy(yout.at[pl.ds(0, TAIL)], y_hbm.at[pl.ds(TAIL_BASE, TAIL)])
        pltpu.sync_copy(effb.at[pl.ds(0, TAIL)], eff_hbm.at[pl.ds(TAIL_BASE, TAIL)])


@jax.jit
def _run(x, v, u, b, beff16):
    mesh = plsc.VectorSubcoreMesh(core_axis_name="c", subcore_axis_name="s",
                                  num_cores=NC, num_subcores=NS)
    eff, y = pl.kernel(
        _body,
        out_type=(jax.ShapeDtypeStruct((N,), jnp.float32),
                  jax.ShapeDtypeStruct((N, DA), jnp.float32)),
        mesh=mesh,
        compiler_params=pltpu.CompilerParams(needs_layout_passes=False),
        scratch_types=(
            pltpu.VMEM((R, DA), jnp.float32),
            pltpu.VMEM((R, DA), jnp.float32),
            pltpu.VMEM((R,), jnp.float32),
            pltpu.VMEM((DA,), jnp.float32),
            pltpu.VMEM((DA,), jnp.float32),
            pltpu.VMEM((DA,), jnp.float32),
            pltpu.VMEM((L,), jnp.float32),
        ),
    )(x, v, u, b, beff16)
    return eff, y


def kernel(x, n, W_eff, b_eff, w_t, b_t):
    v = n * W_eff[0]
    u = w_t * n
    beff16 = jnp.broadcast_to(b_eff[0], (L,))
    eff, y = _run(x, v, u, b_t, beff16)
    return (eff.reshape(N, 1), y)
